# baseline (speedup 1.0000x reference)
import functools

import jax
import jax.numpy as jnp
from jax import lax
from jax.experimental import pallas as pl
from jax.experimental.pallas import tpu as pltpu
from jax.experimental.pallas import tpu_sc as plsc

N = 10000
E = 320000
D = 128
K = 10000

NC = 2
NS = 16
NP = 10240
CH = 128
EPT = 10240
EP = EPT * NC * NS
NCH = EPT // CH
PAIRS = NCH // 2
KPT = NP // (NC * NS)
SCH = 80
RPT = NP // NS
ZR = 64


def _sc_stage(table, keys_pad, src, dst, zeros_blk):
    mesh = plsc.VectorSubcoreMesh(
        core_axis_name="c", subcore_axis_name="s",
        num_cores=NC, num_subcores=NS)

    @functools.partial(
        pl.kernel,
        out_type=[
            jax.ShapeDtypeStruct((NP, D), jnp.float32),
            jax.ShapeDtypeStruct((NC, NP, D), jnp.float32),
        ],
        mesh=mesh,
        scratch_types=[
            pltpu.VMEM((CH,), jnp.int32),
            pltpu.VMEM((CH,), jnp.int32),
            pltpu.VMEM((CH,), jnp.int32),
            pltpu.VMEM((CH,), jnp.int32),
            pltpu.VMEM((CH, D), jnp.float32),
            pltpu.VMEM((CH, D), jnp.float32),
            pltpu.VMEM((ZR, D), jnp.float32),
            pltpu.VMEM_SHARED((NP, D), jnp.float32),
            pltpu.SemaphoreType.DMA,
            pltpu.SemaphoreType.DMA,
            pltpu.SemaphoreType.DMA,
            pltpu.SemaphoreType.DMA,
        ],
    )
    def sc_kernel(table_hbm, keys_hbm, src_hbm, dst_hbm, z_hbm,
                  gs_hbm, parts_hbm,
                  idx_a, idx_b, dst_a, dst_b, rows_a, rows_b, zb_v, acc_sh,
                  gsem_a, gsem_b, ssem_a, ssem_b):
        cid = lax.axis_index("c")
        sid = lax.axis_index("s")
        wid = cid * NS + sid

        kbase = wid * KPT
        for b in range(KPT // SCH):
            iv = idx_a.at[pl.ds(0, SCH)]
            rv = rows_a.at[pl.ds(0, SCH)]
            pltpu.sync_copy(keys_hbm.at[pl.ds(kbase + b * SCH, SCH)], iv)
            pltpu.async_copy(table_hbm.at[iv], rv, gsem_a).wait()
            pltpu.sync_copy(rv, gs_hbm.at[pl.ds(kbase + b * SCH, SCH)])

        pltpu.sync_copy(z_hbm, zb_v)
        zbase = sid * RPT
        for b in range(RPT // ZR):
            pltpu.sync_copy(zb_v, acc_sh.at[pl.ds(zbase + b * ZR, ZR)])
        plsc.subcore_barrier()

        ebase = cid * (EP // NC) + sid * EPT

        def pref(c, iv, dv, rv, gsem):
            base = ebase + c * CH
            pltpu.sync_copy(src_hbm.at[pl.ds(base, CH)], iv)
            pltpu.sync_copy(dst_hbm.at[pl.ds(base, CH)], dv)
            pltpu.async_copy(table_hbm.at[iv], rv, gsem)

        def waitg(iv, rv, gsem):
            pltpu.make_async_copy(table_hbm.at[iv], rv, gsem).wait()

        def scat(dv, rv, ssem):
            pltpu.async_copy(rv, acc_sh.at[dv], ssem, add=True)

        def waits(dv, rv, ssem):
            pltpu.make_async_copy(rv, acc_sh.at[dv], ssem).wait()

        pref(0, idx_a, dst_a, rows_a, gsem_a)
        pref(1, idx_b, dst_b, rows_b, gsem_b)

        def body(p, carry):
            waitg(idx_a, rows_a, gsem_a)
            scat(dst_a, rows_a, ssem_a)
            waitg(idx_b, rows_b, gsem_b)
            scat(dst_b, rows_b, ssem_b)

            @pl.when(p < PAIRS - 1)
            def _():
                waits(dst_a, rows_a, ssem_a)
                pref(2 * p + 2, idx_a, dst_a, rows_a, gsem_a)
                waits(dst_b, rows_b, ssem_b)
                pref(2 * p + 3, idx_b, dst_b, rows_b, gsem_b)

            return carry

        lax.fori_loop(0, PAIRS, body, 0)
        waits(dst_a, rows_a, ssem_a)
        waits(dst_b, rows_b, ssem_b)
        plsc.subcore_barrier()

        for b in range(RPT // CH):
            pltpu.sync_copy(acc_sh.at[pl.ds(zbase + b * CH, CH)], rows_a)
            pltpu.sync_copy(
                rows_a, parts_hbm.at[cid].at[pl.ds(zbase + b * CH, CH)])

    return sc_kernel(table, keys_pad, src, dst, zeros_blk)


BLK = 2000
GRID = N // BLK


def _tc_body(gs_ref, p0_ref, p1_ref, w_ref, m_ref, out_ref, acc_ref):
    i = pl.program_id(0)

    @pl.when(i == 0)
    def _():
        acc_ref[...] = jnp.zeros_like(acc_ref)

    dn = (((1,), (1,)), ((), ()))
    x = lax.dot_general(gs_ref[...], w_ref[...], dn,
                        preferred_element_type=jnp.float32)
    x += lax.dot_general(p0_ref[...] + p1_ref[...], m_ref[...], dn,
                         preferred_element_type=jnp.float32)
    s = jnp.maximum(x, 0.0)
    acc_ref[...] += jnp.sum(s, axis=0, keepdims=True)

    @pl.when(i == GRID - 1)
    def _():
        a = acc_ref[...]
        e = jnp.exp(a - jnp.max(a))
        out_ref[...] = e / jnp.sum(e)


def _tc_stage(gs, p0, p1, W, M):
    return pl.pallas_call(
        _tc_body,
        grid=(GRID,),
        in_specs=[
            pl.BlockSpec((BLK, D), lambda i: (i, 0)),
            pl.BlockSpec((BLK, D), lambda i: (i, 0)),
            pl.BlockSpec((BLK, D), lambda i: (i, 0)),
            pl.BlockSpec((D, D), lambda i: (0, 0)),
            pl.BlockSpec((D, D), lambda i: (0, 0)),
        ],
        out_specs=pl.BlockSpec((1, D), lambda i: (0, 0)),
        out_shape=jax.ShapeDtypeStruct((1, D), jnp.float32),
        scratch_shapes=[pltpu.VMEM((1, D), jnp.float32)],
    )(gs, p0, p1, W, M)


def kernel(index, node_keys, edge_index, W, M, Internal_Node_Impact):
    del index
    pad = EP - E
    src = jnp.concatenate(
        [edge_index[0].astype(jnp.int32), jnp.zeros((pad,), jnp.int32)])
    dst = jnp.concatenate(
        [edge_index[1].astype(jnp.int32),
         N + (jnp.arange(pad, dtype=jnp.int32) % (NP - N))])
    keys_pad = jnp.concatenate(
        [node_keys.astype(jnp.int32), jnp.zeros((NP - N,), jnp.int32)])
    zeros_blk = jnp.zeros((ZR, D), jnp.float32)
    gs, parts = _sc_stage(Internal_Node_Impact, keys_pad, src, dst, zeros_blk)
    out = _tc_stage(gs, parts[0], parts[1], W, M)
    return out.reshape(D, 1)


# speedup vs baseline: 1.5820x; 1.5820x over previous
import functools

import jax
import jax.numpy as jnp
from jax import lax
from jax.experimental import pallas as pl
from jax.experimental.pallas import tpu as pltpu
from jax.experimental.pallas import tpu_sc as plsc

N = 10000
E = 320000
D = 128
K = 10000

NC = 2
NS = 16
NP = 10240
CH = 80
EPT = E // (NC * NS)
NCH = EPT // CH
KPT = NP // (NC * NS)
RPT = NP // NS
ZR = 64
QBITS = 22


def _sc_stage(table_f32, table_q, keys_pad, src, dst, zeros_blk):
    mesh = plsc.VectorSubcoreMesh(
        core_axis_name="c", subcore_axis_name="s",
        num_cores=NC, num_subcores=NS)

    @functools.partial(
        pl.kernel,
        out_type=[
            jax.ShapeDtypeStruct((NP, D), jnp.float32),
            jax.ShapeDtypeStruct((NC, NP, D), jnp.int32),
        ],
        mesh=mesh,
        scratch_types=[
            pltpu.VMEM((CH,), jnp.int32),
            pltpu.VMEM((CH,), jnp.int32),
            pltpu.VMEM((CH, D), jnp.float32),
            pltpu.VMEM((CH, D), jnp.int32),
            pltpu.VMEM((ZR, D), jnp.int32),
            pltpu.VMEM_SHARED((NP, D), jnp.int32),
            pltpu.SemaphoreType.DMA,
        ],
    )
    def sc_kernel(tablef_hbm, tableq_hbm, keys_hbm, src_hbm, dst_hbm, z_hbm,
                  gs_hbm, parts_hbm,
                  idx_v, dst_v, rowsf_v, rowsq_v, zb_v, acc_sh, sem):
        cid = lax.axis_index("c")
        sid = lax.axis_index("s")
        wid = cid * NS + sid

        kbase = wid * KPT
        for b in range(KPT // CH):
            pltpu.sync_copy(keys_hbm.at[pl.ds(kbase + b * CH, CH)], idx_v)
            pltpu.async_copy(tablef_hbm.at[idx_v], rowsf_v, sem).wait()
            pltpu.sync_copy(rowsf_v, gs_hbm.at[pl.ds(kbase + b * CH, CH)])

        pltpu.sync_copy(z_hbm, zb_v)
        zbase = sid * RPT
        for b in range(RPT // ZR):
            pltpu.sync_copy(zb_v, acc_sh.at[pl.ds(zbase + b * ZR, ZR)])
        plsc.subcore_barrier()

        ebase = cid * (E // NC) + sid * EPT

        def body(i, carry):
            base = ebase + i * CH
            pltpu.sync_copy(src_hbm.at[pl.ds(base, CH)], idx_v)
            pltpu.async_copy(tableq_hbm.at[idx_v], rowsq_v, sem).wait()
            pltpu.sync_copy(dst_hbm.at[pl.ds(base, CH)], dst_v)
            pltpu.sync_copy(rowsq_v, acc_sh.at[dst_v], add=True)
            return carry

        lax.fori_loop(0, NCH, body, 0)
        plsc.subcore_barrier()

        for b in range(RPT // CH):
            pltpu.sync_copy(acc_sh.at[pl.ds(zbase + b * CH, CH)], rowsq_v)
            pltpu.sync_copy(
                rowsq_v, parts_hbm.at[cid].at[pl.ds(zbase + b * CH, CH)])

    return sc_kernel(table_f32, table_q, keys_pad, src, dst, zeros_blk)


BLK = 2000
GRID = N // BLK


def _tc_body(gs_ref, p0_ref, p1_ref, w_ref, m_ref, out_ref, acc_ref):
    i = pl.program_id(0)

    @pl.when(i == 0)
    def _():
        acc_ref[...] = jnp.zeros_like(acc_ref)

    dn = (((1,), (1,)), ((), ()))
    x = lax.dot_general(gs_ref[...], w_ref[...], dn,
                        preferred_element_type=jnp.float32,
                        precision=lax.Precision.HIGHEST)
    agg = (p0_ref[...] + p1_ref[...]).astype(jnp.float32) * (0.5 ** QBITS)
    x += lax.dot_general(agg, m_ref[...], dn,
                         preferred_element_type=jnp.float32,
                         precision=lax.Precision.HIGHEST)
    s = jnp.maximum(x, 0.0)
    acc_ref[...] += jnp.sum(s, axis=0, keepdims=True)

    @pl.when(i == GRID - 1)
    def _():
        a = acc_ref[...]
        e = jnp.exp(a - jnp.max(a))
        out_ref[...] = e / jnp.sum(e)


def _tc_stage(gs, p0, p1, W, M):
    return pl.pallas_call(
        _tc_body,
        grid=(GRID,),
        in_specs=[
            pl.BlockSpec((BLK, D), lambda i: (i, 0)),
            pl.BlockSpec((BLK, D), lambda i: (i, 0)),
            pl.BlockSpec((BLK, D), lambda i: (i, 0)),
            pl.BlockSpec((D, D), lambda i: (0, 0)),
            pl.BlockSpec((D, D), lambda i: (0, 0)),
        ],
        out_specs=pl.BlockSpec((1, D), lambda i: (0, 0)),
        out_shape=jax.ShapeDtypeStruct((1, D), jnp.float32),
        scratch_shapes=[pltpu.VMEM((1, D), jnp.float32)],
    )(gs, p0, p1, W, M)


def _round_bf16(x):
    v = lax.bitcast_convert_type(x, jnp.int32)
    r = (v + 0x7FFF + ((v >> 16) & 1)) & ~0xFFFF
    return lax.bitcast_convert_type(r, jnp.float32)


def kernel(index, node_keys, edge_index, W, M, Internal_Node_Impact):
    del index
    src = edge_index[0].astype(jnp.int32)
    dst = edge_index[1].astype(jnp.int32)
    keys_pad = jnp.concatenate(
        [node_keys.astype(jnp.int32), jnp.zeros((NP - N,), jnp.int32)])
    t16 = _round_bf16(Internal_Node_Impact)
    w16 = _round_bf16(W)
    m16 = _round_bf16(M)
    table_q = jnp.round(t16 * (2.0 ** QBITS)).astype(jnp.int32)
    zeros_blk = jnp.zeros((ZR, D), jnp.int32)
    gs, parts = _sc_stage(t16, table_q, keys_pad, src, dst, zeros_blk)
    out = _tc_stage(gs, parts[0], parts[1], w16, m16)
    return out.reshape(D, 1)
